# Initial kernel scaffold; baseline (speedup 1.0000x reference)
import functools

import jax
import jax.numpy as jnp
from jax import lax
from jax.experimental import pallas as pl
from jax.experimental.pallas import tpu as pltpu
from jax.experimental.pallas import tpu_sc as plsc

_GROUP_NUM = 1024
_K = 32

_BIGF = jnp.float32(3.0e38)
_BIGI = jnp.int32(1 << 30)


def _fps_body(ns, xt_ref, sel_ref):
    x = xt_ref[0]
    y = xt_ref[1]
    z = xt_ref[2]
    R, N = x.shape
    iota_p = lax.broadcasted_iota(jnp.int32, (R, N), 1)
    iota_s = lax.broadcasted_iota(jnp.int32, (R, ns), 1)

    d0 = (x * x + y * y) + z * z
    m0 = jnp.max(d0, axis=1, keepdims=True)
    cur0 = jnp.min(jnp.where(d0 == m0, iota_p, _BIGI), axis=1, keepdims=True)

    def body(s, state):
        sel_acc, mind, cur = state
        sel_acc = jnp.where(iota_s == s, cur, sel_acc)
        oh = iota_p == cur
        cx = jnp.sum(jnp.where(oh, x, 0.0), axis=1, keepdims=True)
        cy = jnp.sum(jnp.where(oh, y, 0.0), axis=1, keepdims=True)
        cz = jnp.sum(jnp.where(oh, z, 0.0), axis=1, keepdims=True)
        dx = x - cx
        dy = y - cy
        dz = z - cz
        sq = (dx * dx + dy * dy) + dz * dz
        mind = jnp.minimum(mind, sq)
        mmax = jnp.max(mind, axis=1, keepdims=True)
        cur = jnp.min(jnp.where(mind == mmax, iota_p, _BIGI), axis=1,
                      keepdims=True)
        return sel_acc, mind, cur

    sel0 = jnp.zeros((R, ns), jnp.int32)
    mind0 = jnp.full((R, N), 1e10, jnp.float32)
    sel_acc, _, _ = lax.fori_loop(0, ns, body, (sel0, mind0, cur0))
    sel_ref[...] = sel_acc


def _run_fps(xt, ns):
    R = xt.shape[1]
    return pl.pallas_call(
        functools.partial(_fps_body, ns),
        out_shape=jax.ShapeDtypeStruct((R, ns), jnp.int32),
    )(xt)


def _knn_body(k, q_ref, pt_ref, nn_ref, qp_ref, ppt_ref, ds_ref):
    q3 = q_ref[0, 0]
    S = q3.shape[0]
    px = pt_ref[0, 0:1, :]
    py = pt_ref[0, 1:2, :]
    pz = pt_ref[0, 2:3, :]
    N = px.shape[1]
    G = S // 8

    qp_ref[...] = jnp.zeros_like(qp_ref)
    ppt_ref[...] = jnp.zeros_like(ppt_ref)
    qp_ref[:, 0:3] = q3
    ppt_ref[0:3, :] = pt_ref[0]

    g = lax.dot_general(qp_ref[...], ppt_ref[...],
                        (((1,), (0,)), ((), ())),
                        preferred_element_type=jnp.float32)
    qs = jnp.sum(q3 * q3, axis=1, keepdims=True)
    ps = (px * px + py * py) + pz * pz
    d = jnp.maximum((qs + ps) - 2.0 * g, 0.0)
    ds_ref[...] = d.reshape(G, 8, N)

    iota_p = lax.broadcasted_iota(jnp.int32, (8, N), 1)
    iota_k = lax.broadcasted_iota(jnp.int32, (8, k), 1)

    def group(gi, _):
        db = ds_ref[gi]
        acc = jnp.zeros((8, k), jnp.int32)
        for kk in range(k):
            m = jnp.min(db, axis=1, keepdims=True)
            cand = jnp.where(db == m, iota_p, _BIGI)
            idx = jnp.min(cand, axis=1, keepdims=True)
            db = jnp.where(cand == idx, _BIGF, db)
            acc = jnp.where(iota_k == kk, idx, acc)
        nn_ref[0, gi] = acc
        return 0

    lax.fori_loop(0, G, group, 0)


def _run_knn(q, pt, k):
    R, S = q.shape[0], q.shape[1]
    N = pt.shape[2]
    G = S // 8
    out = pl.pallas_call(
        functools.partial(_knn_body, k),
        grid=(R,),
        in_specs=[
            pl.BlockSpec((1, 1, S, 3), lambda g: (g, 0, 0, 0)),
            pl.BlockSpec((1, 3, N), lambda g: (g, 0, 0)),
        ],
        out_specs=pl.BlockSpec((1, G, 8, k), lambda g: (g, 0, 0, 0)),
        out_shape=jax.ShapeDtypeStruct((R, G, 8, k), jnp.int32),
        scratch_shapes=[
            pltpu.VMEM((S, 128), jnp.float32),
            pltpu.VMEM((128, N), jnp.float32),
            pltpu.VMEM((G, 8, N), jnp.float32),
        ],
        compiler_params=pltpu.CompilerParams(
            dimension_semantics=("arbitrary",)),
    )(q.reshape(R, 1, S, 3), pt)
    return out.reshape(R, S, k)


def _sc_gather(featf, xyzf, idx):
    info = plsc.get_sparse_core_info()
    nw = info.num_cores * info.num_subcores
    rws = idx.shape[0]
    per_w = rws // nw
    chunk = 128 if per_w >= 128 else per_w
    n_chunks = per_w // chunk
    fd = featf.shape[1]
    xd = xyzf.shape[1]
    mesh = plsc.VectorSubcoreMesh(core_axis_name="c", subcore_axis_name="s")

    @functools.partial(
        pl.kernel,
        mesh=mesh,
        out_type=[
            jax.ShapeDtypeStruct((rws, fd), jnp.float32),
            jax.ShapeDtypeStruct((rws, xd), jnp.float32),
        ],
        scratch_types=[
            pltpu.VMEM((chunk,), jnp.int32),
            pltpu.VMEM((chunk, fd), jnp.float32),
            pltpu.VMEM((chunk, xd), jnp.float32),
            pltpu.SemaphoreType.DMA,
        ],
    )
    def gather_k(feat_hbm, xyz_hbm, idx_hbm, feat_out, xyz_out,
                 idx_v, fbuf, xbuf, sem):
        wid = lax.axis_index("s") * info.num_cores + lax.axis_index("c")
        base = wid * per_w

        def body(i, _):
            row0 = base + i * chunk
            pltpu.sync_copy(idx_hbm.at[pl.ds(row0, chunk)], idx_v)
            pltpu.async_copy(feat_hbm.at[idx_v], fbuf, sem).wait()
            pltpu.sync_copy(fbuf, feat_out.at[pl.ds(row0, chunk)])
            pltpu.async_copy(xyz_hbm.at[idx_v], xbuf, sem).wait()
            pltpu.sync_copy(xbuf, xyz_out.at[pl.ds(row0, chunk)])
            return 0

        lax.fori_loop(0, n_chunks, body, 0)

    return gather_k(featf, xyzf, idx)


def kernel(xyz, feat):
    B, N, _ = xyz.shape
    half = N // 2
    C = feat.shape[2]
    ns = min(_GROUP_NUM // 2, max(1, half))
    R = 2 * B

    xh = jnp.concatenate([xyz[:, :half], xyz[:, half:]], axis=0)
    xt = xh.transpose(2, 0, 1)
    sel = _run_fps(xt, ns)

    boff = (jnp.arange(B, dtype=jnp.int32) * N)[:, None, None]
    doff = (jnp.arange(2, dtype=jnp.int32) * half)[None, :, None]
    sel_bd = sel.reshape(2, B, ns).transpose(1, 0, 2)
    lc_idx = (sel_bd + boff + doff).reshape(-1)

    featf = feat.reshape(B * N, C)
    xyzf = xyz.reshape(B * N, 3)
    lc_feat_flat, lc_xyz_flat = _sc_gather(featf, xyzf, lc_idx)

    q = lc_xyz_flat.reshape(B, 2, ns, 3).transpose(1, 0, 2, 3) \
        .reshape(R, ns, 3)
    pt_other = jnp.concatenate([xyz[:, half:], xyz[:, :half]], axis=0) \
        .transpose(0, 2, 1)
    nn = _run_knn(q, pt_other, _K)

    nn_bd = nn.reshape(2, B, ns, _K).transpose(1, 0, 2, 3)
    other_off = (jnp.arange(2, dtype=jnp.int32)[::-1] * half)[None, :, None,
                                                              None]
    nn_idx = (nn_bd + boff[..., None] + other_off).reshape(-1)
    grp_feat_flat, grp_xyz_flat = _sc_gather(featf, xyzf, nn_idx)

    lc_xyz = lc_xyz_flat.reshape(B, 2 * ns, 3)
    lc_feat = lc_feat_flat.reshape(B, 2 * ns, C)
    knn_xyz = grp_xyz_flat.reshape(B, 2 * ns, _K, 3)
    knn_feat = jnp.concatenate(
        [grp_feat_flat.reshape(B, 2 * ns, _K, C), knn_xyz], axis=-1)
    return (lc_xyz, lc_feat, knn_xyz, knn_feat)


# speedup vs baseline: 3.6020x; 3.6020x over previous
import functools

import jax
import jax.numpy as jnp
from jax import lax
from jax.experimental import pallas as pl
from jax.experimental.pallas import tpu as pltpu
from jax.experimental.pallas import tpu_sc as plsc

_GROUP_NUM = 1024
_K = 32

import numpy as np

_BIGF = np.float32(3.0e38)
_BIGI = np.int32(1 << 30)


def _fps_body(ns, xt_ref, sel_ref, q_ref):
    x = xt_ref[0]
    y = xt_ref[1]
    z = xt_ref[2]
    R, N = x.shape
    iota_p = lax.broadcasted_iota(jnp.int32, (R, N), 1)
    iota_s = lax.broadcasted_iota(jnp.int32, (R, ns), 1)

    d0 = (x * x + y * y) + z * z
    m0 = jnp.max(d0, axis=1, keepdims=True)
    cur0 = jnp.min(jnp.where(d0 == m0, iota_p, _BIGI), axis=1, keepdims=True)

    def body(s, state):
        sel_acc, qx, qy, qz, mind, cur = state
        sel_acc = jnp.where(iota_s == s, cur, sel_acc)
        oh = iota_p == cur
        cx = jnp.sum(jnp.where(oh, x, 0.0), axis=1, keepdims=True)
        cy = jnp.sum(jnp.where(oh, y, 0.0), axis=1, keepdims=True)
        cz = jnp.sum(jnp.where(oh, z, 0.0), axis=1, keepdims=True)
        at_s = iota_s == s
        qx = jnp.where(at_s, cx, qx)
        qy = jnp.where(at_s, cy, qy)
        qz = jnp.where(at_s, cz, qz)
        dx = x - cx
        dy = y - cy
        dz = z - cz
        sq = (dx * dx + dy * dy) + dz * dz
        mind = jnp.minimum(mind, sq)
        mmax = jnp.max(mind, axis=1, keepdims=True)
        cur = jnp.min(jnp.where(mind == mmax, iota_p, _BIGI), axis=1,
                      keepdims=True)
        return sel_acc, qx, qy, qz, mind, cur

    sel0 = jnp.zeros((R, ns), jnp.int32)
    q0 = jnp.zeros((R, ns), jnp.float32)
    mind0 = jnp.full((R, N), 1e10, jnp.float32)
    sel_acc, qx, qy, qz, _, _ = lax.fori_loop(
        0, ns, body, (sel0, q0, q0, q0, mind0, cur0))
    sel_ref[...] = sel_acc
    q_ref[0] = qx
    q_ref[1] = qy
    q_ref[2] = qz


def _run_fps(xt, ns):
    R = xt.shape[1]
    return pl.pallas_call(
        functools.partial(_fps_body, ns),
        out_shape=[
            jax.ShapeDtypeStruct((R, ns), jnp.int32),
            jax.ShapeDtypeStruct((3, R, ns), jnp.float32),
        ],
    )(xt)


def _knn_body(k, q_ref, pt_ref, nn_ref, qp_ref, ppt_ref, ds_ref):
    q3 = q_ref[0, 0]
    S = q3.shape[0]
    px = pt_ref[0, 0:1, :]
    py = pt_ref[0, 1:2, :]
    pz = pt_ref[0, 2:3, :]
    N = px.shape[1]
    G = S // 8

    qp_ref[...] = jnp.zeros_like(qp_ref)
    ppt_ref[...] = jnp.zeros_like(ppt_ref)
    qp_ref[:, 0:3] = q3
    ppt_ref[0:3, :] = pt_ref[0]

    g = lax.dot_general(qp_ref[...], ppt_ref[...],
                        (((1,), (0,)), ((), ())),
                        preferred_element_type=jnp.float32)
    qs = jnp.sum(q3 * q3, axis=1, keepdims=True)
    ps = (px * px + py * py) + pz * pz
    d = jnp.maximum((qs + ps) - 2.0 * g, 0.0)
    ds_ref[...] = d.reshape(G, 8, N)

    iota_p = lax.broadcasted_iota(jnp.int32, (8, N), 1)
    iota_k = lax.broadcasted_iota(jnp.int32, (8, k), 1)

    def group(gi, _):
        db = ds_ref[gi]
        acc = jnp.zeros((8, k), jnp.int32)
        for kk in range(k):
            m = jnp.min(db, axis=1, keepdims=True)
            cand = jnp.where(db == m, iota_p, _BIGI)
            idx = jnp.min(cand, axis=1, keepdims=True)
            db = jnp.where(cand == idx, _BIGF, db)
            acc = jnp.where(iota_k == kk, idx, acc)
        nn_ref[0, gi] = acc
        return 0

    lax.fori_loop(0, G, group, 0)


def _run_knn(q, pt, k):
    R, S = q.shape[0], q.shape[1]
    N = pt.shape[2]
    G = S // 8
    out = pl.pallas_call(
        functools.partial(_knn_body, k),
        grid=(R,),
        in_specs=[
            pl.BlockSpec((1, 1, S, 3), lambda g: (g, 0, 0, 0)),
            pl.BlockSpec((1, 3, N), lambda g: (g, 0, 0)),
        ],
        out_specs=pl.BlockSpec((1, G, 8, k), lambda g: (g, 0, 0, 0)),
        out_shape=jax.ShapeDtypeStruct((R, G, 8, k), jnp.int32),
        scratch_shapes=[
            pltpu.VMEM((S, 128), jnp.float32),
            pltpu.VMEM((128, N), jnp.float32),
            pltpu.VMEM((G, 8, N), jnp.float32),
        ],
        compiler_params=pltpu.CompilerParams(
            dimension_semantics=("arbitrary",)),
    )(q.reshape(R, 1, S, 3), pt)
    return out.reshape(R, S, k)


def _sc_gather(featf, xyzflat, idx):
    info = plsc.get_sparse_core_info()
    nw = info.num_cores * info.num_subcores
    rws = idx.shape[0]
    per_w = rws // nw
    chunk = 128 if per_w >= 128 else per_w
    n_chunks = per_w // chunk
    fd = featf.shape[1]
    with_xyz = xyzflat is not None
    mesh = plsc.VectorSubcoreMesh(core_axis_name="c", subcore_axis_name="s")

    out_type = [jax.ShapeDtypeStruct((rws, fd), jnp.float32)]
    scratch = [
        pltpu.VMEM((chunk,), jnp.int32),
        pltpu.VMEM((chunk, fd), jnp.float32),
        pltpu.SemaphoreType.DMA,
    ]
    if with_xyz:
        for _ in range(3):
            out_type.append(jax.ShapeDtypeStruct((rws,), jnp.float32))
        scratch.append(pltpu.VMEM((chunk,), jnp.int32))
        scratch.append(pltpu.VMEM((chunk,), jnp.float32))

    @functools.partial(pl.kernel, mesh=mesh, out_type=out_type,
                       scratch_types=scratch)
    def gather_k(*refs):
        if with_xyz:
            (feat_hbm, xyz_hbm, idx_hbm, feat_out, xo0, xo1, xo2,
             idx_v, fbuf, sem, idxb, xbuf) = refs
            xyz_out = (xo0, xo1, xo2)
        else:
            feat_hbm, idx_hbm, feat_out, idx_v, fbuf, sem = refs
        wid = lax.axis_index("s") * info.num_cores + lax.axis_index("c")
        base = wid * per_w

        def body(i, _):
            row0 = base + i * chunk
            pltpu.sync_copy(idx_hbm.at[pl.ds(row0, chunk)], idx_v)
            pltpu.async_copy(feat_hbm.at[idx_v], fbuf, sem).wait()
            pltpu.sync_copy(fbuf, feat_out.at[pl.ds(row0, chunk)])
            if with_xyz:
                for c in range(3):
                    for j in range(chunk // 16):
                        sl = pl.ds(j * 16, 16)
                        idxb[sl] = idx_v[sl] * 3 + c
                    pltpu.async_copy(xyz_hbm.at[idxb], xbuf, sem).wait()
                    pltpu.sync_copy(xbuf, xyz_out[c].at[pl.ds(row0, chunk)])
            return 0

        lax.fori_loop(0, n_chunks, body, 0)

    return gather_k(*((featf, xyzflat, idx) if with_xyz else (featf, idx)))


def kernel(xyz, feat):
    B, N, _ = xyz.shape
    half = N // 2
    C = feat.shape[2]
    ns = min(_GROUP_NUM // 2, max(1, half))
    R = 2 * B

    xh = jnp.concatenate([xyz[:, :half], xyz[:, half:]], axis=0)
    xt = xh.transpose(2, 0, 1)
    sel, qxyz = _run_fps(xt, ns)

    boff = (jnp.arange(B, dtype=jnp.int32) * N)[:, None, None]
    doff = (jnp.arange(2, dtype=jnp.int32) * half)[None, :, None]
    sel_bd = sel.reshape(2, B, ns).transpose(1, 0, 2)
    lc_idx = (sel_bd + boff + doff).reshape(-1)

    featf = feat.reshape(B * N, C)
    (lc_feat_flat,) = _sc_gather(featf, None, lc_idx)

    q = qxyz.transpose(1, 2, 0)
    pt_other = jnp.concatenate([xyz[:, half:], xyz[:, :half]], axis=0) \
        .transpose(0, 2, 1)
    nn = _run_knn(q, pt_other, _K)

    nn_bd = nn.reshape(2, B, ns, _K).transpose(1, 0, 2, 3)
    other_off = (jnp.arange(2, dtype=jnp.int32)[::-1] * half)[None, :, None,
                                                              None]
    nn_idx = (nn_bd + boff[..., None] + other_off).reshape(-1)
    grp_feat_flat, gx, gy, gz = _sc_gather(featf, xyz.reshape(-1), nn_idx)

    lc_xyz = q.reshape(2, B, ns, 3).transpose(1, 0, 2, 3).reshape(B, 2 * ns, 3)
    lc_feat = lc_feat_flat.reshape(B, 2 * ns, C)
    knn_xyz = jnp.stack([gx, gy, gz], axis=-1).reshape(B, 2 * ns, _K, 3)
    knn_feat = jnp.concatenate(
        [grp_feat_flat.reshape(B, 2 * ns, _K, C), knn_xyz], axis=-1)
    return (lc_xyz, lc_feat, knn_xyz, knn_feat)


# speedup vs baseline: 9.3244x; 2.5887x over previous
import functools

import jax
import jax.numpy as jnp
from jax import lax
from jax.experimental import pallas as pl
from jax.experimental.pallas import tpu as pltpu
from jax.experimental.pallas import tpu_sc as plsc

_GROUP_NUM = 1024
_K = 32

import numpy as np

_BIGF = np.float32(3.0e38)
_BIGI = np.int32(1 << 30)


def _fps_body(ns, xt_ref, sel_ref, q_ref):
    x = xt_ref[0]
    y = xt_ref[1]
    z = xt_ref[2]
    R, N = x.shape
    iota_p = lax.broadcasted_iota(jnp.int32, (R, N), 1)
    iota_s = lax.broadcasted_iota(jnp.int32, (R, ns), 1)

    d0 = (x * x + y * y) + z * z
    m0 = jnp.max(d0, axis=1, keepdims=True)
    cur0 = jnp.min(jnp.where(d0 == m0, iota_p, _BIGI), axis=1, keepdims=True)

    def body(s, state):
        sel_acc, qx, qy, qz, mind, cur = state
        sel_acc = jnp.where(iota_s == s, cur, sel_acc)
        oh = iota_p == cur
        cx = jnp.sum(jnp.where(oh, x, 0.0), axis=1, keepdims=True)
        cy = jnp.sum(jnp.where(oh, y, 0.0), axis=1, keepdims=True)
        cz = jnp.sum(jnp.where(oh, z, 0.0), axis=1, keepdims=True)
        at_s = iota_s == s
        qx = jnp.where(at_s, cx, qx)
        qy = jnp.where(at_s, cy, qy)
        qz = jnp.where(at_s, cz, qz)
        dx = x - cx
        dy = y - cy
        dz = z - cz
        sq = (dx * dx + dy * dy) + dz * dz
        mind = jnp.minimum(mind, sq)
        mmax = jnp.max(mind, axis=1, keepdims=True)
        cur = jnp.min(jnp.where(mind == mmax, iota_p, _BIGI), axis=1,
                      keepdims=True)
        return sel_acc, qx, qy, qz, mind, cur

    sel0 = jnp.zeros((R, ns), jnp.int32)
    q0 = jnp.zeros((R, ns), jnp.float32)
    mind0 = jnp.full((R, N), 1e10, jnp.float32)
    sel_acc, qx, qy, qz, _, _ = lax.fori_loop(
        0, ns, body, (sel0, q0, q0, q0, mind0, cur0))
    sel_ref[...] = sel_acc
    q_ref[0] = qx
    q_ref[1] = qy
    q_ref[2] = qz


def _run_fps(xt, ns):
    R = xt.shape[1]
    return pl.pallas_call(
        functools.partial(_fps_body, ns),
        out_shape=[
            jax.ShapeDtypeStruct((R, ns), jnp.int32),
            jax.ShapeDtypeStruct((3, R, ns), jnp.float32),
        ],
    )(xt)


def _knn_body(k, q_ref, pt_ref, nn_ref, qp_ref, ppt_ref, ds_ref):
    q3 = q_ref[0, 0]
    S = q3.shape[0]
    px = pt_ref[0, 0:1, :]
    py = pt_ref[0, 1:2, :]
    pz = pt_ref[0, 2:3, :]
    N = px.shape[1]
    RG = 32 if S % 32 == 0 else 8
    G = S // RG

    qp_ref[...] = jnp.zeros_like(qp_ref)
    ppt_ref[...] = jnp.zeros_like(ppt_ref)
    qp_ref[:, 0:3] = q3
    ppt_ref[0:3, :] = pt_ref[0]

    g = lax.dot_general(qp_ref[...], ppt_ref[...],
                        (((1,), (0,)), ((), ())),
                        preferred_element_type=jnp.float32)
    qs = jnp.sum(q3 * q3, axis=1, keepdims=True)
    ps = (px * px + py * py) + pz * pz
    d = jnp.maximum((qs + ps) - 2.0 * g, 0.0)
    ds_ref[...] = d.reshape(G, RG, N)

    iota_p = lax.broadcasted_iota(jnp.int32, (RG, N), 1)
    iota_k = lax.broadcasted_iota(jnp.int32, (RG, k), 1)

    def group(gi, _):
        db = ds_ref[gi]
        acc = jnp.zeros((RG, k), jnp.int32)
        for kk in range(k):
            m = jnp.min(db, axis=1, keepdims=True)
            cand = jnp.where(db == m, iota_p, _BIGI)
            idx = jnp.min(cand, axis=1, keepdims=True)
            db = jnp.where(cand == idx, _BIGF, db)
            acc = jnp.where(iota_k == kk, idx, acc)
        nn_ref[0, gi] = acc
        return 0

    lax.fori_loop(0, G, group, 0)


def _run_knn(q, pt, k):
    R, S = q.shape[0], q.shape[1]
    N = pt.shape[2]
    RG = 32 if S % 32 == 0 else 8
    G = S // RG
    out = pl.pallas_call(
        functools.partial(_knn_body, k),
        grid=(R,),
        in_specs=[
            pl.BlockSpec((1, 1, S, 3), lambda g: (g, 0, 0, 0)),
            pl.BlockSpec((1, 3, N), lambda g: (g, 0, 0)),
        ],
        out_specs=pl.BlockSpec((1, G, RG, k), lambda g: (g, 0, 0, 0)),
        out_shape=jax.ShapeDtypeStruct((R, G, RG, k), jnp.int32),
        scratch_shapes=[
            pltpu.VMEM((S, 128), jnp.float32),
            pltpu.VMEM((128, N), jnp.float32),
            pltpu.VMEM((G, RG, N), jnp.float32),
        ],
        compiler_params=pltpu.CompilerParams(
            dimension_semantics=("arbitrary",)),
    )(q.reshape(R, 1, S, 3), pt)
    return out.reshape(R, S, k)


def _sc_gather(featf, xyzflat, idx):
    info = plsc.get_sparse_core_info()
    nw = info.num_cores * info.num_subcores
    rws = idx.shape[0]
    per_w = rws // nw
    chunk = 128 if per_w >= 128 else per_w
    n_chunks = per_w // chunk
    fd = featf.shape[1]
    with_xyz = xyzflat is not None
    mesh = plsc.VectorSubcoreMesh(core_axis_name="c", subcore_axis_name="s")

    out_type = [jax.ShapeDtypeStruct((rws, fd), jnp.float32)]
    scratch = [
        pltpu.VMEM((chunk,), jnp.int32),
        pltpu.VMEM((chunk, fd), jnp.float32),
        pltpu.SemaphoreType.DMA,
    ]
    if with_xyz:
        for _ in range(3):
            out_type.append(jax.ShapeDtypeStruct((rws,), jnp.float32))
        scratch.append(pltpu.VMEM((chunk,), jnp.int32))
        scratch.append(pltpu.VMEM((chunk,), jnp.float32))

    @functools.partial(pl.kernel, mesh=mesh, out_type=out_type,
                       scratch_types=scratch)
    def gather_k(*refs):
        if with_xyz:
            (feat_hbm, xyz_hbm, idx_hbm, feat_out, xo0, xo1, xo2,
             idx_v, fbuf, sem, idxb, xbuf) = refs
            xyz_out = (xo0, xo1, xo2)
        else:
            feat_hbm, idx_hbm, feat_out, idx_v, fbuf, sem = refs
        wid = lax.axis_index("s") * info.num_cores + lax.axis_index("c")
        base = wid * per_w

        def body(i, _):
            row0 = base + i * chunk
            pltpu.sync_copy(idx_hbm.at[pl.ds(row0, chunk)], idx_v)
            pltpu.async_copy(feat_hbm.at[idx_v], fbuf, sem).wait()
            pltpu.sync_copy(fbuf, feat_out.at[pl.ds(row0, chunk)])
            if with_xyz:
                for c in range(3):
                    for j in range(chunk // 16):
                        sl = pl.ds(j * 16, 16)
                        idxb[sl] = idx_v[sl] * 3 + c
                    pltpu.async_copy(xyz_hbm.at[idxb], xbuf, sem).wait()
                    pltpu.sync_copy(xbuf, xyz_out[c].at[pl.ds(row0, chunk)])
            return 0

        lax.fori_loop(0, n_chunks, body, 0)

    return gather_k(*((featf, xyzflat, idx) if with_xyz else (featf, idx)))


def kernel(xyz, feat):
    B, N, _ = xyz.shape
    half = N // 2
    C = feat.shape[2]
    ns = min(_GROUP_NUM // 2, max(1, half))
    R = 2 * B

    xh = jnp.concatenate([xyz[:, :half], xyz[:, half:]], axis=0)
    xt = xh.transpose(2, 0, 1)
    sel, qxyz = _run_fps(xt, ns)

    boff = (jnp.arange(B, dtype=jnp.int32) * N)[:, None, None]
    doff = (jnp.arange(2, dtype=jnp.int32) * half)[None, :, None]
    sel_bd = sel.reshape(2, B, ns).transpose(1, 0, 2)
    lc_idx = (sel_bd + boff + doff).reshape(-1)

    featf = feat.reshape(B * N, C)
    (lc_feat_flat,) = _sc_gather(featf, None, lc_idx)

    q = qxyz.transpose(1, 2, 0)
    pt_other = jnp.concatenate([xyz[:, half:], xyz[:, :half]], axis=0) \
        .transpose(0, 2, 1)
    nn = _run_knn(q, pt_other, _K)

    nn_bd = nn.reshape(2, B, ns, _K).transpose(1, 0, 2, 3)
    other_off = (jnp.arange(2, dtype=jnp.int32)[::-1] * half)[None, :, None,
                                                              None]
    nn_idx = (nn_bd + boff[..., None] + other_off).reshape(-1)
    grp_feat_flat, gx, gy, gz = _sc_gather(featf, xyz.reshape(-1), nn_idx)

    lc_xyz = q.reshape(2, B, ns, 3).transpose(1, 0, 2, 3).reshape(B, 2 * ns, 3)
    lc_feat = lc_feat_flat.reshape(B, 2 * ns, C)
    knn_xyz = jnp.stack([gx, gy, gz], axis=-1).reshape(B, 2 * ns, _K, 3)
    knn_feat = jnp.concatenate(
        [grp_feat_flat.reshape(B, 2 * ns, _K, C), knn_xyz], axis=-1)
    return (lc_xyz, lc_feat, knn_xyz, knn_feat)


# speedup vs baseline: 12.4519x; 1.3354x over previous
import functools

import jax
import jax.numpy as jnp
from jax import lax
from jax.experimental import pallas as pl
from jax.experimental.pallas import tpu as pltpu
from jax.experimental.pallas import tpu_sc as plsc

_GROUP_NUM = 1024
_K = 32

import numpy as np

_BIGF = np.float32(3.0e38)
_BIGI = np.int32(1 << 30)


def _fps_body(ns, xt_ref, sel_ref, q_ref):
    x = xt_ref[0]
    y = xt_ref[1]
    z = xt_ref[2]
    R, N = x.shape
    iota_p = lax.broadcasted_iota(jnp.int32, (R, N), 1)
    iota_s = lax.broadcasted_iota(jnp.int32, (R, ns), 1)

    d0 = (x * x + y * y) + z * z
    m0 = jnp.max(d0, axis=1, keepdims=True)
    cur0 = jnp.min(jnp.where(d0 == m0, iota_p, _BIGI), axis=1, keepdims=True)

    def body(s, state):
        sel_acc, qx, qy, qz, mind, cur = state
        sel_acc = jnp.where(iota_s == s, cur, sel_acc)
        oh = iota_p == cur
        cx = jnp.sum(jnp.where(oh, x, 0.0), axis=1, keepdims=True)
        cy = jnp.sum(jnp.where(oh, y, 0.0), axis=1, keepdims=True)
        cz = jnp.sum(jnp.where(oh, z, 0.0), axis=1, keepdims=True)
        at_s = iota_s == s
        qx = jnp.where(at_s, cx, qx)
        qy = jnp.where(at_s, cy, qy)
        qz = jnp.where(at_s, cz, qz)
        dx = x - cx
        dy = y - cy
        dz = z - cz
        sq = (dx * dx + dy * dy) + dz * dz
        mind = jnp.minimum(mind, sq)
        mmax = jnp.max(mind, axis=1, keepdims=True)
        cur = jnp.min(jnp.where(mind == mmax, iota_p, _BIGI), axis=1,
                      keepdims=True)
        return sel_acc, qx, qy, qz, mind, cur

    sel0 = jnp.zeros((R, ns), jnp.int32)
    q0 = jnp.zeros((R, ns), jnp.float32)
    mind0 = jnp.full((R, N), 1e10, jnp.float32)
    sel_acc, qx, qy, qz, _, _ = lax.fori_loop(
        0, ns, body, (sel0, q0, q0, q0, mind0, cur0))
    sel_ref[...] = sel_acc
    q_ref[0] = qx
    q_ref[1] = qy
    q_ref[2] = qz


def _run_fps(xt, ns):
    R = xt.shape[1]
    return pl.pallas_call(
        functools.partial(_fps_body, ns),
        out_shape=[
            jax.ShapeDtypeStruct((R, ns), jnp.int32),
            jax.ShapeDtypeStruct((3, R, ns), jnp.float32),
        ],
    )(xt)


def _knn_body(k, q_ref, pt_ref, nn_ref, qp_ref, ppt_ref, ds_ref):
    q3 = q_ref[0, 0]
    S = q3.shape[0]
    px = pt_ref[0, 0:1, :]
    py = pt_ref[0, 1:2, :]
    pz = pt_ref[0, 2:3, :]
    N = px.shape[1]
    RG = 64 if S % 64 == 0 else 8
    G = S // RG

    qp_ref[...] = jnp.zeros_like(qp_ref)
    ppt_ref[...] = jnp.zeros_like(ppt_ref)
    qp_ref[:, 0:3] = q3
    ppt_ref[0:3, :] = pt_ref[0]

    g = lax.dot_general(qp_ref[...], ppt_ref[...],
                        (((1,), (0,)), ((), ())),
                        preferred_element_type=jnp.float32)
    qs = jnp.sum(q3 * q3, axis=1, keepdims=True)
    ps = (px * px + py * py) + pz * pz
    d = jnp.maximum((qs + ps) - 2.0 * g, 0.0)
    ds_ref[...] = d.reshape(G, RG, N)

    iota_p = lax.broadcasted_iota(jnp.int32, (RG, N), 1)
    iota_k = lax.broadcasted_iota(jnp.int32, (RG, k), 1)

    def group(gi, _):
        db = ds_ref[gi]
        acc = jnp.zeros((RG, k), jnp.int32)
        for kk in range(k):
            m = jnp.min(db, axis=1, keepdims=True)
            cand = jnp.where(db == m, iota_p, _BIGI)
            idx = jnp.min(cand, axis=1, keepdims=True)
            db = jnp.where(cand == idx, _BIGF, db)
            acc = jnp.where(iota_k == kk, idx, acc)
        nn_ref[0, gi] = acc
        return 0

    lax.fori_loop(0, G, group, 0)


def _run_knn(q, pt, k):
    R, S = q.shape[0], q.shape[1]
    N = pt.shape[2]
    RG = 64 if S % 64 == 0 else 8
    G = S // RG
    out = pl.pallas_call(
        functools.partial(_knn_body, k),
        grid=(R,),
        in_specs=[
            pl.BlockSpec((1, 1, S, 3), lambda g: (g, 0, 0, 0)),
            pl.BlockSpec((1, 3, N), lambda g: (g, 0, 0)),
        ],
        out_specs=pl.BlockSpec((1, G, RG, k), lambda g: (g, 0, 0, 0)),
        out_shape=jax.ShapeDtypeStruct((R, G, RG, k), jnp.int32),
        scratch_shapes=[
            pltpu.VMEM((S, 128), jnp.float32),
            pltpu.VMEM((128, N), jnp.float32),
            pltpu.VMEM((G, RG, N), jnp.float32),
        ],
        compiler_params=pltpu.CompilerParams(
            dimension_semantics=("arbitrary",)),
    )(q.reshape(R, 1, S, 3), pt)
    return out.reshape(R, S, k)


def _sc_gather(featf, xyzflat, idx):
    info = plsc.get_sparse_core_info()
    nw = info.num_cores * info.num_subcores
    rws = idx.shape[0]
    per_w = rws // nw
    chunk = 128 if per_w >= 128 else per_w
    n_chunks = per_w // chunk
    fd = featf.shape[1]
    with_xyz = xyzflat is not None
    mesh = plsc.VectorSubcoreMesh(core_axis_name="c", subcore_axis_name="s")

    out_type = [jax.ShapeDtypeStruct((rws, fd), jnp.float32)]
    scratch = [
        pltpu.VMEM((chunk,), jnp.int32),
        pltpu.VMEM((chunk, fd), jnp.float32),
        pltpu.SemaphoreType.DMA,
    ]
    if with_xyz:
        for _ in range(3):
            out_type.append(jax.ShapeDtypeStruct((rws,), jnp.float32))
        scratch.append(pltpu.VMEM((chunk,), jnp.int32))
        scratch.append(pltpu.VMEM((chunk,), jnp.float32))

    @functools.partial(pl.kernel, mesh=mesh, out_type=out_type,
                       scratch_types=scratch)
    def gather_k(*refs):
        if with_xyz:
            (feat_hbm, xyz_hbm, idx_hbm, feat_out, xo0, xo1, xo2,
             idx_v, fbuf, sem, idxb, xbuf) = refs
            xyz_out = (xo0, xo1, xo2)
        else:
            feat_hbm, idx_hbm, feat_out, idx_v, fbuf, sem = refs
        wid = lax.axis_index("s") * info.num_cores + lax.axis_index("c")
        base = wid * per_w

        def body(i, _):
            row0 = base + i * chunk
            pltpu.sync_copy(idx_hbm.at[pl.ds(row0, chunk)], idx_v)
            pltpu.async_copy(feat_hbm.at[idx_v], fbuf, sem).wait()
            pltpu.sync_copy(fbuf, feat_out.at[pl.ds(row0, chunk)])
            if with_xyz:
                for c in range(3):
                    for j in range(chunk // 16):
                        sl = pl.ds(j * 16, 16)
                        idxb[sl] = idx_v[sl] * 3 + c
                    pltpu.async_copy(xyz_hbm.at[idxb], xbuf, sem).wait()
                    pltpu.sync_copy(xbuf, xyz_out[c].at[pl.ds(row0, chunk)])
            return 0

        lax.fori_loop(0, n_chunks, body, 0)

    return gather_k(*((featf, xyzflat, idx) if with_xyz else (featf, idx)))


def kernel(xyz, feat):
    B, N, _ = xyz.shape
    half = N // 2
    C = feat.shape[2]
    ns = min(_GROUP_NUM // 2, max(1, half))
    R = 2 * B

    xh = jnp.concatenate([xyz[:, :half], xyz[:, half:]], axis=0)
    xt = xh.transpose(2, 0, 1)
    sel, qxyz = _run_fps(xt, ns)

    boff = (jnp.arange(B, dtype=jnp.int32) * N)[:, None, None]
    doff = (jnp.arange(2, dtype=jnp.int32) * half)[None, :, None]
    sel_bd = sel.reshape(2, B, ns).transpose(1, 0, 2)
    lc_idx = (sel_bd + boff + doff).reshape(-1)

    featf = feat.reshape(B * N, C)
    (lc_feat_flat,) = _sc_gather(featf, None, lc_idx)

    q = qxyz.transpose(1, 2, 0)
    pt_other = jnp.concatenate([xyz[:, half:], xyz[:, :half]], axis=0) \
        .transpose(0, 2, 1)
    nn = _run_knn(q, pt_other, _K)

    nn_bd = nn.reshape(2, B, ns, _K).transpose(1, 0, 2, 3)
    other_off = (jnp.arange(2, dtype=jnp.int32)[::-1] * half)[None, :, None,
                                                              None]
    nn_idx = (nn_bd + boff[..., None] + other_off).reshape(-1)
    grp_feat_flat, gx, gy, gz = _sc_gather(featf, xyz.reshape(-1), nn_idx)

    lc_xyz = q.reshape(2, B, ns, 3).transpose(1, 0, 2, 3).reshape(B, 2 * ns, 3)
    lc_feat = lc_feat_flat.reshape(B, 2 * ns, C)
    knn_xyz = jnp.stack([gx, gy, gz], axis=-1).reshape(B, 2 * ns, _K, 3)
    knn_feat = jnp.concatenate(
        [grp_feat_flat.reshape(B, 2 * ns, _K, C), knn_xyz], axis=-1)
    return (lc_xyz, lc_feat, knn_xyz, knn_feat)


# speedup vs baseline: 13.5585x; 1.0889x over previous
import functools

import jax
import jax.numpy as jnp
from jax import lax
from jax.experimental import pallas as pl
from jax.experimental.pallas import tpu as pltpu
from jax.experimental.pallas import tpu_sc as plsc

_GROUP_NUM = 1024
_K = 32

import numpy as np

_BIGF = np.float32(3.0e38)
_BIGI = np.int32(1 << 30)


def _fps_body(ns, xt_ref, sel_ref, q_ref):
    x = xt_ref[0]
    y = xt_ref[1]
    z = xt_ref[2]
    R, N = x.shape
    iota_p = lax.broadcasted_iota(jnp.int32, (R, N), 1)
    iota_s = lax.broadcasted_iota(jnp.int32, (R, ns), 1)

    d0 = (x * x + y * y) + z * z
    m0 = jnp.max(d0, axis=1, keepdims=True)
    cur0 = jnp.min(jnp.where(d0 == m0, iota_p, _BIGI), axis=1, keepdims=True)

    def body(s, state):
        sel_acc, qx, qy, qz, mind, cur = state
        sel_acc = jnp.where(iota_s == s, cur, sel_acc)
        oh = iota_p == cur
        cx = jnp.sum(jnp.where(oh, x, 0.0), axis=1, keepdims=True)
        cy = jnp.sum(jnp.where(oh, y, 0.0), axis=1, keepdims=True)
        cz = jnp.sum(jnp.where(oh, z, 0.0), axis=1, keepdims=True)
        at_s = iota_s == s
        qx = jnp.where(at_s, cx, qx)
        qy = jnp.where(at_s, cy, qy)
        qz = jnp.where(at_s, cz, qz)
        dx = x - cx
        dy = y - cy
        dz = z - cz
        sq = (dx * dx + dy * dy) + dz * dz
        mind = jnp.minimum(mind, sq)
        mmax = jnp.max(mind, axis=1, keepdims=True)
        cur = jnp.min(jnp.where(mind == mmax, iota_p, _BIGI), axis=1,
                      keepdims=True)
        return sel_acc, qx, qy, qz, mind, cur

    sel0 = jnp.zeros((R, ns), jnp.int32)
    q0 = jnp.zeros((R, ns), jnp.float32)
    mind0 = jnp.full((R, N), 1e10, jnp.float32)
    sel_acc, qx, qy, qz, _, _ = lax.fori_loop(
        0, ns, body, (sel0, q0, q0, q0, mind0, cur0))
    sel_ref[...] = sel_acc
    q_ref[0] = qx
    q_ref[1] = qy
    q_ref[2] = qz


def _run_fps(xt, ns):
    R = xt.shape[1]
    return pl.pallas_call(
        functools.partial(_fps_body, ns),
        out_shape=[
            jax.ShapeDtypeStruct((R, ns), jnp.int32),
            jax.ShapeDtypeStruct((3, R, ns), jnp.float32),
        ],
    )(xt)


def _knn_body(k, q_ref, pt_ref, nn_ref, qp_ref, ppt_ref, ds_ref):
    q3 = q_ref[0, 0]
    S = q3.shape[0]
    px = pt_ref[0, 0:1, :]
    py = pt_ref[0, 1:2, :]
    pz = pt_ref[0, 2:3, :]
    N = px.shape[1]
    RG = 128 if S % 128 == 0 else 8
    G = S // RG

    qp_ref[...] = jnp.zeros_like(qp_ref)
    ppt_ref[...] = jnp.zeros_like(ppt_ref)
    qp_ref[:, 0:3] = q3
    ppt_ref[0:3, :] = pt_ref[0]

    g = lax.dot_general(qp_ref[...], ppt_ref[...],
                        (((1,), (0,)), ((), ())),
                        preferred_element_type=jnp.float32)
    qs = jnp.sum(q3 * q3, axis=1, keepdims=True)
    ps = (px * px + py * py) + pz * pz
    d = jnp.maximum((qs + ps) - 2.0 * g, 0.0)
    ds_ref[...] = d.reshape(G, RG, N)

    iota_p = lax.broadcasted_iota(jnp.int32, (RG, N), 1)
    iota_k = lax.broadcasted_iota(jnp.int32, (RG, k), 1)

    def group(gi, _):
        db = ds_ref[gi]
        acc = jnp.zeros((RG, k), jnp.int32)
        for kk in range(k):
            m = jnp.min(db, axis=1, keepdims=True)
            cand = jnp.where(db == m, iota_p, _BIGI)
            idx = jnp.min(cand, axis=1, keepdims=True)
            db = jnp.where(cand == idx, _BIGF, db)
            acc = jnp.where(iota_k == kk, idx, acc)
        nn_ref[0, gi] = acc
        return 0

    lax.fori_loop(0, G, group, 0)


def _run_knn(q, pt, k):
    R, S = q.shape[0], q.shape[1]
    N = pt.shape[2]
    RG = 128 if S % 128 == 0 else 8
    G = S // RG
    out = pl.pallas_call(
        functools.partial(_knn_body, k),
        grid=(R,),
        in_specs=[
            pl.BlockSpec((1, 1, S, 3), lambda g: (g, 0, 0, 0)),
            pl.BlockSpec((1, 3, N), lambda g: (g, 0, 0)),
        ],
        out_specs=pl.BlockSpec((1, G, RG, k), lambda g: (g, 0, 0, 0)),
        out_shape=jax.ShapeDtypeStruct((R, G, RG, k), jnp.int32),
        scratch_shapes=[
            pltpu.VMEM((S, 128), jnp.float32),
            pltpu.VMEM((128, N), jnp.float32),
            pltpu.VMEM((G, RG, N), jnp.float32),
        ],
        compiler_params=pltpu.CompilerParams(
            dimension_semantics=("arbitrary",)),
    )(q.reshape(R, 1, S, 3), pt)
    return out.reshape(R, S, k)


def _sc_gather(featf, xyzflat, idx):
    info = plsc.get_sparse_core_info()
    nw = info.num_cores * info.num_subcores
    rws = idx.shape[0]
    per_w = rws // nw
    chunk = 128 if per_w >= 128 else per_w
    n_chunks = per_w // chunk
    fd = featf.shape[1]
    with_xyz = xyzflat is not None
    mesh = plsc.VectorSubcoreMesh(core_axis_name="c", subcore_axis_name="s")

    out_type = [jax.ShapeDtypeStruct((rws, fd), jnp.float32)]
    scratch = [
        pltpu.VMEM((chunk,), jnp.int32),
        pltpu.VMEM((chunk, fd), jnp.float32),
        pltpu.SemaphoreType.DMA,
    ]
    if with_xyz:
        for _ in range(3):
            out_type.append(jax.ShapeDtypeStruct((rws,), jnp.float32))
        scratch.append(pltpu.VMEM((chunk,), jnp.int32))
        scratch.append(pltpu.VMEM((chunk,), jnp.float32))

    @functools.partial(pl.kernel, mesh=mesh, out_type=out_type,
                       scratch_types=scratch)
    def gather_k(*refs):
        if with_xyz:
            (feat_hbm, xyz_hbm, idx_hbm, feat_out, xo0, xo1, xo2,
             idx_v, fbuf, sem, idxb, xbuf) = refs
            xyz_out = (xo0, xo1, xo2)
        else:
            feat_hbm, idx_hbm, feat_out, idx_v, fbuf, sem = refs
        wid = lax.axis_index("s") * info.num_cores + lax.axis_index("c")
        base = wid * per_w

        def body(i, _):
            row0 = base + i * chunk
            pltpu.sync_copy(idx_hbm.at[pl.ds(row0, chunk)], idx_v)
            pltpu.async_copy(feat_hbm.at[idx_v], fbuf, sem).wait()
            pltpu.sync_copy(fbuf, feat_out.at[pl.ds(row0, chunk)])
            if with_xyz:
                for c in range(3):
                    for j in range(chunk // 16):
                        sl = pl.ds(j * 16, 16)
                        idxb[sl] = idx_v[sl] * 3 + c
                    pltpu.async_copy(xyz_hbm.at[idxb], xbuf, sem).wait()
                    pltpu.sync_copy(xbuf, xyz_out[c].at[pl.ds(row0, chunk)])
            return 0

        lax.fori_loop(0, n_chunks, body, 0)

    return gather_k(*((featf, xyzflat, idx) if with_xyz else (featf, idx)))


def kernel(xyz, feat):
    B, N, _ = xyz.shape
    half = N // 2
    C = feat.shape[2]
    ns = min(_GROUP_NUM // 2, max(1, half))
    R = 2 * B

    xh = jnp.concatenate([xyz[:, :half], xyz[:, half:]], axis=0)
    xt = xh.transpose(2, 0, 1)
    sel, qxyz = _run_fps(xt, ns)

    boff = (jnp.arange(B, dtype=jnp.int32) * N)[:, None, None]
    doff = (jnp.arange(2, dtype=jnp.int32) * half)[None, :, None]
    sel_bd = sel.reshape(2, B, ns).transpose(1, 0, 2)
    lc_idx = (sel_bd + boff + doff).reshape(-1)

    featf = feat.reshape(B * N, C)
    (lc_feat_flat,) = _sc_gather(featf, None, lc_idx)

    q = qxyz.transpose(1, 2, 0)
    pt_other = jnp.concatenate([xyz[:, half:], xyz[:, :half]], axis=0) \
        .transpose(0, 2, 1)
    nn = _run_knn(q, pt_other, _K)

    nn_bd = nn.reshape(2, B, ns, _K).transpose(1, 0, 2, 3)
    other_off = (jnp.arange(2, dtype=jnp.int32)[::-1] * half)[None, :, None,
                                                              None]
    nn_idx = (nn_bd + boff[..., None] + other_off).reshape(-1)
    grp_feat_flat, gx, gy, gz = _sc_gather(featf, xyz.reshape(-1), nn_idx)

    lc_xyz = q.reshape(2, B, ns, 3).transpose(1, 0, 2, 3).reshape(B, 2 * ns, 3)
    lc_feat = lc_feat_flat.reshape(B, 2 * ns, C)
    knn_xyz = jnp.stack([gx, gy, gz], axis=-1).reshape(B, 2 * ns, _K, 3)
    knn_feat = jnp.concatenate(
        [grp_feat_flat.reshape(B, 2 * ns, _K, C), knn_xyz], axis=-1)
    return (lc_xyz, lc_feat, knn_xyz, knn_feat)


# speedup vs baseline: 14.0429x; 1.0357x over previous
import functools

import jax
import jax.numpy as jnp
from jax import lax
from jax.experimental import pallas as pl
from jax.experimental.pallas import tpu as pltpu
from jax.experimental.pallas import tpu_sc as plsc

_GROUP_NUM = 1024
_K = 32

import numpy as np

_BIGF = np.float32(3.0e38)
_BIGI = np.int32(1 << 30)


def _fps_body(ns, xt_ref, sel_ref, q_ref):
    x = xt_ref[0]
    y = xt_ref[1]
    z = xt_ref[2]
    R, N = x.shape
    iota_p = lax.broadcasted_iota(jnp.int32, (R, N), 1)
    iota_s = lax.broadcasted_iota(jnp.int32, (R, ns), 1)

    d0 = (x * x + y * y) + z * z
    m0 = jnp.max(d0, axis=1, keepdims=True)
    cur0 = jnp.min(jnp.where(d0 == m0, iota_p, _BIGI), axis=1, keepdims=True)

    def body(s, state):
        sel_acc, qx, qy, qz, mind, cur = state
        sel_acc = jnp.where(iota_s == s, cur, sel_acc)
        oh = iota_p == cur
        cx = jnp.sum(jnp.where(oh, x, 0.0), axis=1, keepdims=True)
        cy = jnp.sum(jnp.where(oh, y, 0.0), axis=1, keepdims=True)
        cz = jnp.sum(jnp.where(oh, z, 0.0), axis=1, keepdims=True)
        at_s = iota_s == s
        qx = jnp.where(at_s, cx, qx)
        qy = jnp.where(at_s, cy, qy)
        qz = jnp.where(at_s, cz, qz)
        dx = x - cx
        dy = y - cy
        dz = z - cz
        sq = (dx * dx + dy * dy) + dz * dz
        mind = jnp.minimum(mind, sq)
        mmax = jnp.max(mind, axis=1, keepdims=True)
        cur = jnp.min(jnp.where(mind == mmax, iota_p, _BIGI), axis=1,
                      keepdims=True)
        return sel_acc, qx, qy, qz, mind, cur

    sel0 = jnp.zeros((R, ns), jnp.int32)
    q0 = jnp.zeros((R, ns), jnp.float32)
    mind0 = jnp.full((R, N), 1e10, jnp.float32)
    sel_acc, qx, qy, qz, _, _ = lax.fori_loop(
        0, ns, body, (sel0, q0, q0, q0, mind0, cur0))
    sel_ref[...] = sel_acc
    q_ref[0] = qx
    q_ref[1] = qy
    q_ref[2] = qz


def _run_fps(xt, ns):
    R = xt.shape[1]
    return pl.pallas_call(
        functools.partial(_fps_body, ns),
        out_shape=[
            jax.ShapeDtypeStruct((R, ns), jnp.int32),
            jax.ShapeDtypeStruct((3, R, ns), jnp.float32),
        ],
    )(xt)


def _knn_body(k, q_ref, pt_ref, nn_ref, qp_ref, ppt_ref, ds_ref):
    q3 = q_ref[0, 0]
    S = q3.shape[0]
    px = pt_ref[0, 0:1, :]
    py = pt_ref[0, 1:2, :]
    pz = pt_ref[0, 2:3, :]
    N = px.shape[1]
    RG = 128 if S % 128 == 0 else 8
    G = S // RG

    qp_ref[...] = jnp.zeros_like(qp_ref)
    ppt_ref[...] = jnp.zeros_like(ppt_ref)
    qp_ref[:, 0:3] = q3
    ppt_ref[0:3, :] = pt_ref[0]

    g = lax.dot_general(qp_ref[...], ppt_ref[...],
                        (((1,), (0,)), ((), ())),
                        preferred_element_type=jnp.float32)
    qs = jnp.sum(q3 * q3, axis=1, keepdims=True)
    ps = (px * px + py * py) + pz * pz
    d = jnp.maximum((qs + ps) - 2.0 * g, 0.0)
    ds_ref[...] = d.reshape(G, RG, N)

    iota_p = lax.broadcasted_iota(jnp.int32, (RG, N), 1)
    iota_k = lax.broadcasted_iota(jnp.int32, (RG, k), 1)

    def group(gi, _):
        db = ds_ref[gi]
        acc = jnp.zeros((RG, k), jnp.int32)
        for kk in range(k):
            m = jnp.min(db, axis=1, keepdims=True)
            cand = jnp.where(db == m, iota_p, _BIGI)
            idx = jnp.min(cand, axis=1, keepdims=True)
            db = jnp.where(cand == idx, _BIGF, db)
            acc = jnp.where(iota_k == kk, idx, acc)
        nn_ref[0, gi] = acc
        return 0

    lax.fori_loop(0, G, group, 0)


def _run_knn(q, pt, k):
    R, S = q.shape[0], q.shape[1]
    N = pt.shape[2]
    RG = 128 if S % 128 == 0 else 8
    G = S // RG
    out = pl.pallas_call(
        functools.partial(_knn_body, k),
        grid=(R,),
        in_specs=[
            pl.BlockSpec((1, 1, S, 3), lambda g: (g, 0, 0, 0)),
            pl.BlockSpec((1, 3, N), lambda g: (g, 0, 0)),
        ],
        out_specs=pl.BlockSpec((1, G, RG, k), lambda g: (g, 0, 0, 0)),
        out_shape=jax.ShapeDtypeStruct((R, G, RG, k), jnp.int32),
        scratch_shapes=[
            pltpu.VMEM((S, 128), jnp.float32),
            pltpu.VMEM((128, N), jnp.float32),
            pltpu.VMEM((G, RG, N), jnp.float32),
        ],
        compiler_params=pltpu.CompilerParams(
            dimension_semantics=("arbitrary",)),
    )(q.reshape(R, 1, S, 3), pt)
    return out.reshape(R, S, k)


def _sc_gather(featf, xyzflat, idx):
    info = plsc.get_sparse_core_info()
    nw = info.num_cores * info.num_subcores
    rws = idx.shape[0]
    per_w = rws // nw
    chunk = 128 if per_w >= 128 else per_w
    n_chunks = per_w // chunk
    fd = featf.shape[1]
    with_xyz = xyzflat is not None
    out_fd = fd + 3 if with_xyz else fd
    mesh = plsc.VectorSubcoreMesh(core_axis_name="c", subcore_axis_name="s")

    out_type = [jax.ShapeDtypeStruct((rws, out_fd), jnp.float32)]
    scratch = [
        pltpu.VMEM((chunk,), jnp.int32),
        pltpu.VMEM((chunk, fd), jnp.float32),
        pltpu.SemaphoreType.DMA,
    ]
    if with_xyz:
        for _ in range(3):
            out_type.append(jax.ShapeDtypeStruct((rws,), jnp.float32))
        scratch.append(pltpu.VMEM((chunk,), jnp.int32))
        scratch.append(pltpu.VMEM((chunk,), jnp.float32))

    @functools.partial(pl.kernel, mesh=mesh, out_type=out_type,
                       scratch_types=scratch)
    def gather_k(*refs):
        if with_xyz:
            (feat_hbm, xyz_hbm, idx_hbm, feat_out, xo0, xo1, xo2,
             idx_v, fbuf, sem, idxb, xbuf) = refs
            xyz_out = (xo0, xo1, xo2)
        else:
            feat_hbm, idx_hbm, feat_out, idx_v, fbuf, sem = refs
        wid = lax.axis_index("s") * info.num_cores + lax.axis_index("c")
        base = wid * per_w

        def body(i, _):
            row0 = base + i * chunk
            pltpu.sync_copy(idx_hbm.at[pl.ds(row0, chunk)], idx_v)
            pltpu.async_copy(feat_hbm.at[idx_v], fbuf, sem).wait()
            if with_xyz:
                pltpu.sync_copy(
                    fbuf, feat_out.at[pl.ds(row0, chunk), pl.ds(0, fd)])
            else:
                pltpu.sync_copy(fbuf, feat_out.at[pl.ds(row0, chunk)])
            if with_xyz:
                for c in range(3):
                    for j in range(chunk // 16):
                        sl = pl.ds(j * 16, 16)
                        idxb[sl] = idx_v[sl] * 3 + c
                    pltpu.async_copy(xyz_hbm.at[idxb], xbuf, sem).wait()
                    pltpu.sync_copy(xbuf, xyz_out[c].at[pl.ds(row0, chunk)])
            return 0

        lax.fori_loop(0, n_chunks, body, 0)

    return gather_k(*((featf, xyzflat, idx) if with_xyz else (featf, idx)))


def kernel(xyz, feat):
    B, N, _ = xyz.shape
    half = N // 2
    C = feat.shape[2]
    ns = min(_GROUP_NUM // 2, max(1, half))
    R = 2 * B

    xh = jnp.concatenate([xyz[:, :half], xyz[:, half:]], axis=0)
    xt = xh.transpose(2, 0, 1)
    sel, qxyz = _run_fps(xt, ns)

    boff = (jnp.arange(B, dtype=jnp.int32) * N)[:, None, None]
    doff = (jnp.arange(2, dtype=jnp.int32) * half)[None, :, None]
    sel_bd = sel.reshape(2, B, ns).transpose(1, 0, 2)
    lc_idx = (sel_bd + boff + doff).reshape(-1)

    featf = feat.reshape(B * N, C)
    (lc_feat_flat,) = _sc_gather(featf, None, lc_idx)

    q = qxyz.transpose(1, 2, 0)
    pt_other = jnp.concatenate([xyz[:, half:], xyz[:, :half]], axis=0) \
        .transpose(0, 2, 1)
    nn = _run_knn(q, pt_other, _K)

    nn_bd = nn.reshape(2, B, ns, _K).transpose(1, 0, 2, 3)
    other_off = (jnp.arange(2, dtype=jnp.int32)[::-1] * half)[None, :, None,
                                                              None]
    nn_idx = (nn_bd + boff[..., None] + other_off).reshape(-1)
    grp_feat_flat, gx, gy, gz = _sc_gather(featf, xyz.reshape(-1), nn_idx)

    lc_xyz = q.reshape(2, B, ns, 3).transpose(1, 0, 2, 3).reshape(B, 2 * ns, 3)
    lc_feat = lc_feat_flat.reshape(B, 2 * ns, C)
    knn_xyz = jnp.stack([gx, gy, gz], axis=-1).reshape(B, 2 * ns, _K, 3)
    knn_feat = lax.dynamic_update_slice(
        grp_feat_flat.reshape(B, 2 * ns, _K, C + 3), knn_xyz, (0, 0, 0, C))
    return (lc_xyz, lc_feat, knn_xyz, knn_feat)


# speedup vs baseline: 14.0555x; 1.0009x over previous
import functools

import jax
import jax.numpy as jnp
from jax import lax
from jax.experimental import pallas as pl
from jax.experimental.pallas import tpu as pltpu
from jax.experimental.pallas import tpu_sc as plsc

_GROUP_NUM = 1024
_K = 32

import numpy as np

_BIGF = np.float32(3.0e38)
_BIGI = np.int32(1 << 30)


def _fps_body(ns, xt_ref, sel_ref, q_ref):
    x = xt_ref[0]
    y = xt_ref[1]
    z = xt_ref[2]
    R, N = x.shape
    iota_p = lax.broadcasted_iota(jnp.int32, (R, N), 1)
    iota_s = lax.broadcasted_iota(jnp.int32, (R, ns), 1)

    d0 = (x * x + y * y) + z * z
    m0 = jnp.max(d0, axis=1, keepdims=True)
    cur0 = jnp.min(jnp.where(d0 == m0, iota_p, _BIGI), axis=1, keepdims=True)

    def body(s, state):
        sel_acc, qx, qy, qz, mind, cur = state
        sel_acc = jnp.where(iota_s == s, cur, sel_acc)
        oh = iota_p == cur
        cx = jnp.sum(jnp.where(oh, x, 0.0), axis=1, keepdims=True)
        cy = jnp.sum(jnp.where(oh, y, 0.0), axis=1, keepdims=True)
        cz = jnp.sum(jnp.where(oh, z, 0.0), axis=1, keepdims=True)
        at_s = iota_s == s
        qx = jnp.where(at_s, cx, qx)
        qy = jnp.where(at_s, cy, qy)
        qz = jnp.where(at_s, cz, qz)
        dx = x - cx
        dy = y - cy
        dz = z - cz
        sq = (dx * dx + dy * dy) + dz * dz
        mind = jnp.minimum(mind, sq)
        mmax = jnp.max(mind, axis=1, keepdims=True)
        cur = jnp.min(jnp.where(mind == mmax, iota_p, _BIGI), axis=1,
                      keepdims=True)
        return sel_acc, qx, qy, qz, mind, cur

    sel0 = jnp.zeros((R, ns), jnp.int32)
    q0 = jnp.zeros((R, ns), jnp.float32)
    mind0 = jnp.full((R, N), 1e10, jnp.float32)
    sel_acc, qx, qy, qz, _, _ = lax.fori_loop(
        0, ns, body, (sel0, q0, q0, q0, mind0, cur0))
    sel_ref[...] = sel_acc
    q_ref[0] = qx
    q_ref[1] = qy
    q_ref[2] = qz


def _run_fps(xt, ns):
    R = xt.shape[1]
    return pl.pallas_call(
        functools.partial(_fps_body, ns),
        out_shape=[
            jax.ShapeDtypeStruct((R, ns), jnp.int32),
            jax.ShapeDtypeStruct((3, R, ns), jnp.float32),
        ],
    )(xt)


def _knn_body(k, q_ref, pt_ref, nn_ref, qp_ref, ppt_ref, ds_ref):
    q3 = q_ref[0, 0]
    S = q3.shape[0]
    px = pt_ref[0, 0:1, :]
    py = pt_ref[0, 1:2, :]
    pz = pt_ref[0, 2:3, :]
    N = px.shape[1]
    RG = 128 if S % 128 == 0 else 8
    G = S // RG

    qp_ref[...] = jnp.zeros_like(qp_ref)
    ppt_ref[...] = jnp.zeros_like(ppt_ref)
    qp_ref[:, 0:3] = q3
    ppt_ref[0:3, :] = pt_ref[0]

    g = lax.dot_general(qp_ref[...], ppt_ref[...],
                        (((1,), (0,)), ((), ())),
                        preferred_element_type=jnp.float32)
    qs = jnp.sum(q3 * q3, axis=1, keepdims=True)
    ps = (px * px + py * py) + pz * pz
    d = jnp.maximum((qs + ps) - 2.0 * g, 0.0)
    ds_ref[...] = d.reshape(G, RG, N)

    iota_p = lax.broadcasted_iota(jnp.int32, (RG, N), 1)
    iota_k = lax.broadcasted_iota(jnp.int32, (RG, k), 1)

    def group(gi, _):
        db = ds_ref[gi]
        acc = jnp.zeros((RG, k), jnp.int32)
        for kk in range(k):
            m = jnp.min(db, axis=1, keepdims=True)
            cand = jnp.where(db == m, iota_p, _BIGI)
            idx = jnp.min(cand, axis=1, keepdims=True)
            db = jnp.where(cand == idx, _BIGF, db)
            acc = jnp.where(iota_k == kk, idx, acc)
        nn_ref[0, gi] = acc
        return 0

    lax.fori_loop(0, G, group, 0)


def _run_knn(q, pt, k):
    R, S = q.shape[0], q.shape[1]
    N = pt.shape[2]
    RG = 128 if S % 128 == 0 else 8
    G = S // RG
    out = pl.pallas_call(
        functools.partial(_knn_body, k),
        grid=(R,),
        in_specs=[
            pl.BlockSpec((1, 1, S, 3), lambda g: (g, 0, 0, 0)),
            pl.BlockSpec((1, 3, N), lambda g: (g, 0, 0)),
        ],
        out_specs=pl.BlockSpec((1, G, RG, k), lambda g: (g, 0, 0, 0)),
        out_shape=jax.ShapeDtypeStruct((R, G, RG, k), jnp.int32),
        scratch_shapes=[
            pltpu.VMEM((S, 128), jnp.float32),
            pltpu.VMEM((128, N), jnp.float32),
            pltpu.VMEM((G, RG, N), jnp.float32),
        ],
        compiler_params=pltpu.CompilerParams(
            dimension_semantics=("parallel",)),
    )(q.reshape(R, 1, S, 3), pt)
    return out.reshape(R, S, k)


def _sc_gather(featf, xyzflat, idx):
    info = plsc.get_sparse_core_info()
    nw = info.num_cores * info.num_subcores
    rws = idx.shape[0]
    per_w = rws // nw
    chunk = 128 if per_w >= 128 else per_w
    n_chunks = per_w // chunk
    fd = featf.shape[1]
    with_xyz = xyzflat is not None
    out_fd = fd + 3 if with_xyz else fd
    mesh = plsc.VectorSubcoreMesh(core_axis_name="c", subcore_axis_name="s")

    out_type = [jax.ShapeDtypeStruct((rws, out_fd), jnp.float32)]
    scratch = [
        pltpu.VMEM((chunk,), jnp.int32),
        pltpu.VMEM((chunk, fd), jnp.float32),
        pltpu.SemaphoreType.DMA,
    ]
    if with_xyz:
        for _ in range(3):
            out_type.append(jax.ShapeDtypeStruct((rws,), jnp.float32))
        scratch.append(pltpu.VMEM((chunk,), jnp.int32))
        scratch.append(pltpu.VMEM((chunk,), jnp.float32))

    @functools.partial(pl.kernel, mesh=mesh, out_type=out_type,
                       scratch_types=scratch)
    def gather_k(*refs):
        if with_xyz:
            (feat_hbm, xyz_hbm, idx_hbm, feat_out, xo0, xo1, xo2,
             idx_v, fbuf, sem, idxb, xbuf) = refs
            xyz_out = (xo0, xo1, xo2)
        else:
            feat_hbm, idx_hbm, feat_out, idx_v, fbuf, sem = refs
        wid = lax.axis_index("s") * info.num_cores + lax.axis_index("c")
        base = wid * per_w

        def body(i, _):
            row0 = base + i * chunk
            pltpu.sync_copy(idx_hbm.at[pl.ds(row0, chunk)], idx_v)
            pltpu.async_copy(feat_hbm.at[idx_v], fbuf, sem).wait()
            if with_xyz:
                pltpu.sync_copy(
                    fbuf, feat_out.at[pl.ds(row0, chunk), pl.ds(0, fd)])
            else:
                pltpu.sync_copy(fbuf, feat_out.at[pl.ds(row0, chunk)])
            if with_xyz:
                for c in range(3):
                    for j in range(chunk // 16):
                        sl = pl.ds(j * 16, 16)
                        idxb[sl] = idx_v[sl] * 3 + c
                    pltpu.async_copy(xyz_hbm.at[idxb], xbuf, sem).wait()
                    pltpu.sync_copy(xbuf, xyz_out[c].at[pl.ds(row0, chunk)])
            return 0

        lax.fori_loop(0, n_chunks, body, 0)

    return gather_k(*((featf, xyzflat, idx) if with_xyz else (featf, idx)))


def kernel(xyz, feat):
    B, N, _ = xyz.shape
    half = N // 2
    C = feat.shape[2]
    ns = min(_GROUP_NUM // 2, max(1, half))
    R = 2 * B

    xh = jnp.concatenate([xyz[:, :half], xyz[:, half:]], axis=0)
    xt = xh.transpose(2, 0, 1)
    sel, qxyz = _run_fps(xt, ns)

    boff = (jnp.arange(B, dtype=jnp.int32) * N)[:, None, None]
    doff = (jnp.arange(2, dtype=jnp.int32) * half)[None, :, None]
    sel_bd = sel.reshape(2, B, ns).transpose(1, 0, 2)
    lc_idx = (sel_bd + boff + doff).reshape(-1)

    featf = feat.reshape(B * N, C)
    (lc_feat_flat,) = _sc_gather(featf, None, lc_idx)

    q = qxyz.transpose(1, 2, 0)
    pt_other = jnp.concatenate([xyz[:, half:], xyz[:, :half]], axis=0) \
        .transpose(0, 2, 1)
    nn = _run_knn(q, pt_other, _K)

    nn_bd = nn.reshape(2, B, ns, _K).transpose(1, 0, 2, 3)
    other_off = (jnp.arange(2, dtype=jnp.int32)[::-1] * half)[None, :, None,
                                                              None]
    nn_idx = (nn_bd + boff[..., None] + other_off).reshape(-1)
    grp_feat_flat, gx, gy, gz = _sc_gather(featf, xyz.reshape(-1), nn_idx)

    lc_xyz = q.reshape(2, B, ns, 3).transpose(1, 0, 2, 3).reshape(B, 2 * ns, 3)
    lc_feat = lc_feat_flat.reshape(B, 2 * ns, C)
    knn_xyz = jnp.stack([gx, gy, gz], axis=-1).reshape(B, 2 * ns, _K, 3)
    knn_feat = lax.dynamic_update_slice(
        grp_feat_flat.reshape(B, 2 * ns, _K, C + 3), knn_xyz, (0, 0, 0, C))
    return (lc_xyz, lc_feat, knn_xyz, knn_feat)
